# baseline (speedup 1.0000x reference)
import functools

import jax
import jax.numpy as jnp
from jax import lax
from jax.experimental import pallas as pl
from jax.experimental.pallas import tpu as pltpu
from jax.experimental.pallas import tpu_sc as plsc

NC = 2
NS = 16
NW = NC * NS

BATCH = 16384
EMBED = 32
LANES = 16
PACK = 4
ROWS_PER_W = BATCH // NW
NCH = ROWS_PER_W // 128
NWAVES = NCH * (128 // LANES)
WSLOTS = 3 * LANES


def _sc_gather_body(uid, mid, cid, ut, mt, ct, out, idx_v, wide, buf, sem):
  wid = lax.axis_index("s") * NC + lax.axis_index("c")
  base = wid * NCH
  pltpu.sync_copy(uid.at[pl.ds(base, NCH)], idx_v.at[0])
  pltpu.sync_copy(mid.at[pl.ds(base, NCH)], idx_v.at[1])
  pltpu.sync_copy(cid.at[pl.ds(base, NCH)], idx_v.at[2])
  tables = (ut, mt, ct)

  def ids_of(cc):
    j = cc // (128 // LANES)
    col0 = (cc - j * (128 // LANES)) * LANES
    return j, col0, [idx_v[t, j, pl.ds(col0, LANES)] for t in range(3)]

  def fire(cc):
    _, _, w = ids_of(cc)
    pbase = (cc % 2) * WSLOTS
    for ii in range(LANES):
      for t in range(3):
        line = lax.shift_right_logical(w[t][ii], 2)
        pltpu.async_copy(tables[t].at[line],
                         wide.at[pbase + 3 * ii + t], sem)

  def drain():
    for _ in range(WSLOTS):
      pltpu.make_async_copy(ut.at[0], wide.at[0], sem).wait()

  def extract(cc):
    j, col0, w = ids_of(cc)
    pbase = (cc % 2) * WSLOTS
    for ii in range(LANES):
      col = col0 + ii
      for t in range(3):
        m = lax.bitwise_and(w[t][ii], PACK - 1) * EMBED
        for half in range(2):
          vals = wide[pbase + 3 * ii + t, pl.ds(m + half * LANES, LANES)]
          buf[j, col, pl.ds(t * EMBED + half * LANES, LANES)] = vals

  fire(0)

  def wave_body(cc, _):
    fire(cc + 1)
    drain()
    extract(cc)
    return 0

  lax.fori_loop(0, NWAVES - 1, wave_body, 0)
  drain()
  extract(NWAVES - 1)
  pltpu.sync_copy(buf, out.at[pl.ds(base, NCH)])


@jax.jit
def _sc_gather(uid, mid, cid, ut, mt, ct):
  n = BATCH // 128
  mesh = plsc.VectorSubcoreMesh(
      core_axis_name="c", subcore_axis_name="s",
      num_cores=NC, num_subcores=NS)
  fn = pl.kernel(
      _sc_gather_body,
      out_type=jax.ShapeDtypeStruct((n, 128, 3 * EMBED), jnp.float32),
      mesh=mesh,
      scratch_types=[
          pltpu.VMEM((3, NCH, 128), jnp.int32),
          pltpu.VMEM((2 * WSLOTS, PACK * EMBED), jnp.float32),
          pltpu.VMEM((NCH, 128, 3 * EMBED), jnp.float32),
          pltpu.SemaphoreType.DMA,
      ],
  )
  return fn(uid.reshape(n, 128), mid.reshape(n, 128), cid.reshape(n, 128),
            ut, mt, ct)


def _mlp_body(x, w1, b1, w2, b2, out):
  h = jnp.dot(x[...], w1[...], preferred_element_type=jnp.float32)
  h = jnp.maximum(h + b1[...], 0.0)
  out[...] = jnp.sum(h * w2[...], axis=1, keepdims=True) + b2[...]


@functools.partial(jax.jit, static_argnames=("bs",))
def _mlp(x, w1, b1, w2, b2, bs=2048):
  grid = BATCH // bs
  full = lambda shape: pl.BlockSpec(shape, lambda i: (0,) * len(shape))
  return pl.pallas_call(
      _mlp_body,
      grid=(grid,),
      in_specs=[pl.BlockSpec((bs, 3 * EMBED), lambda i: (i, 0)),
                full((3 * EMBED, 64)), full((1, 64)),
                full((1, 64)), full((1, 1))],
      out_specs=pl.BlockSpec((bs, 1), lambda i: (i, 0)),
      out_shape=jax.ShapeDtypeStruct((BATCH, 1), jnp.float32),
  )(x, w1, b1, w2, b2)


def kernel(user_ids, movie_ids, categories, user_table, movie_table,
           cat_table, W1, b1, W2, b2):
  x = _sc_gather(user_ids.astype(jnp.int32), movie_ids.astype(jnp.int32),
                 categories.astype(jnp.int32),
                 user_table.reshape(-1, PACK * EMBED),
                 movie_table.reshape(-1, PACK * EMBED),
                 cat_table.reshape(-1, PACK * EMBED))
  x = x.reshape(BATCH, 3 * EMBED)
  return _mlp(x, W1, b1.reshape(1, 64), W2.reshape(1, 64), b2.reshape(1, 1))


# speedup vs baseline: 1.5541x; 1.5541x over previous
import functools

import jax
import jax.numpy as jnp
from jax import lax
from jax.experimental import pallas as pl
from jax.experimental.pallas import tpu as pltpu
from jax.experimental.pallas import tpu_sc as plsc

NC = 2
NS = 16
NW = NC * NS

BATCH = 16384
EMBED = 32
LANES = 16
ROWS_PER_W = BATCH // NW
NCH = ROWS_PER_W // 128
NWAVES = NCH * (128 // LANES)


def _sc_gather_body(uid, mid, cid, ut, mt, ct, out, idx_v, buf, sem):
  wid = lax.axis_index("s") * NC + lax.axis_index("c")
  base = wid * NCH
  pltpu.sync_copy(uid.at[pl.ds(base, NCH)], idx_v.at[0])
  pltpu.sync_copy(mid.at[pl.ds(base, NCH)], idx_v.at[1])
  pltpu.sync_copy(cid.at[pl.ds(base, NCH)], idx_v.at[2])
  tables = (ut, mt, ct)

  def fire(cc):
    j = cc // (128 // LANES)
    col0 = (cc - j * (128 // LANES)) * LANES
    w = [idx_v[t, j, pl.ds(col0, LANES)] for t in range(3)]
    for ii in range(LANES):
      col = col0 + ii
      for t in range(3):
        pltpu.async_copy(tables[t].at[w[t][ii]],
                         buf.at[j, col, pl.ds(t * EMBED, EMBED)], sem)

  def drain():
    for ii in range(LANES):
      for t in range(3):
        pltpu.make_async_copy(
            tables[t].at[0],
            buf.at[0, 0, pl.ds(t * EMBED, EMBED)], sem).wait()

  fire(0)

  def wave_body(cc, _):
    fire(cc + 1)
    drain()
    return 0

  lax.fori_loop(0, NWAVES - 1, wave_body, 0)
  drain()
  pltpu.sync_copy(buf, out.at[pl.ds(base, NCH)])


@jax.jit
def _sc_gather(uid, mid, cid, ut, mt, ct):
  n = BATCH // 128
  mesh = plsc.VectorSubcoreMesh(
      core_axis_name="c", subcore_axis_name="s",
      num_cores=NC, num_subcores=NS)
  fn = pl.kernel(
      _sc_gather_body,
      out_type=jax.ShapeDtypeStruct((n, 128, 3 * EMBED), jnp.float32),
      mesh=mesh,
      scratch_types=[
          pltpu.VMEM((3, NCH, 128), jnp.int32),
          pltpu.VMEM((NCH, 128, 3 * EMBED), jnp.float32),
          pltpu.SemaphoreType.DMA,
      ],
  )
  return fn(uid.reshape(n, 128), mid.reshape(n, 128), cid.reshape(n, 128),
            ut, mt, ct)


def _mlp_body(x, w1, b1, w2, b2, out):
  h = jnp.dot(x[...], w1[...], preferred_element_type=jnp.float32)
  h = jnp.maximum(h + b1[...], 0.0)
  out[...] = jnp.sum(h * w2[...], axis=1, keepdims=True) + b2[...]


@functools.partial(jax.jit, static_argnames=("bs",))
def _mlp(x, w1, b1, w2, b2, bs=2048):
  grid = BATCH // bs
  full = lambda shape: pl.BlockSpec(shape, lambda i: (0,) * len(shape))
  return pl.pallas_call(
      _mlp_body,
      grid=(grid,),
      in_specs=[pl.BlockSpec((bs, 3 * EMBED), lambda i: (i, 0)),
                full((3 * EMBED, 64)), full((1, 64)),
                full((1, 64)), full((1, 1))],
      out_specs=pl.BlockSpec((bs, 1), lambda i: (i, 0)),
      out_shape=jax.ShapeDtypeStruct((BATCH, 1), jnp.float32),
  )(x, w1, b1, w2, b2)


def kernel(user_ids, movie_ids, categories, user_table, movie_table,
           cat_table, W1, b1, W2, b2):
  x = _sc_gather(user_ids.astype(jnp.int32), movie_ids.astype(jnp.int32),
                 categories.astype(jnp.int32),
                 user_table, movie_table, cat_table)
  x = x.reshape(BATCH, 3 * EMBED)
  return _mlp(x, W1, b1.reshape(1, 64), W2.reshape(1, 64), b2.reshape(1, 1))
